# Initial kernel scaffold; baseline (speedup 1.0000x reference)
import jax
import jax.numpy as jnp
from jax.experimental import pallas as pl


def kernel(feat, edge_index, etypes, W):
    raise NotImplementedError("write your pallas kernel here")


# speedup vs baseline: 12.3468x; 12.3468x over previous
import functools

import jax
import jax.numpy as jnp
from jax import lax
from jax.experimental import pallas as pl
from jax.experimental.pallas import tpu as pltpu
from jax.experimental.pallas import tpu_sc as plsc

N_NODES = 10000
N_EDGES = 320000
D = 128
R = 8

NC = 1
NS = 16
NW = NC * NS

CHUNK = 80
RB = 10
NBLK = N_EDGES // (CHUNK * RB * NW)
N_PAD = 10240
NODES_PER_TILE = N_PAD // NS


def _mm_body(feat_ref, w_ref, out_ref):
    out_ref[0] = jnp.dot(feat_ref[...], w_ref[0],
                         preferred_element_type=jnp.float32)


def _typed_transform(feat, W):
    BN = 1000
    NB = N_NODES // BN
    return pl.pallas_call(
        _mm_body,
        grid=(NB, R),
        in_specs=[
            pl.BlockSpec((BN, D), lambda n, r: (n, 0)),
            pl.BlockSpec((1, D, D), lambda n, r: (r, 0, 0)),
        ],
        out_specs=pl.BlockSpec((1, BN, D), lambda n, r: (r, n, 0)),
        out_shape=jax.ShapeDtypeStruct((R, N_NODES, D), jnp.float32),
    )(feat, W)


def _sc_body(table, src4, dst4, et4, out, src_v, dst_v, et_v, gidx_v,
             rows_v, shared_acc, sem):
    s = lax.axis_index("s")
    wid = s
    base = s * NODES_PER_TILE

    def _zero_row(j, _):
        for k in range(D // 16):
            rows_v[j, pl.ds(k * 16, 16)] = jnp.zeros((16,), jnp.float32)
        return 0
    lax.fori_loop(0, CHUNK, _zero_row, 0)
    for i in range(NODES_PER_TILE // CHUNK):
        pltpu.sync_copy(rows_v, shared_acc.at[pl.ds(base + i * CHUNK, CHUNK)])

    plsc.subcore_barrier()

    def _block(b, _):
        pltpu.sync_copy(src4.at[wid, b], src_v)
        pltpu.sync_copy(dst4.at[wid, b], dst_v)
        pltpu.sync_copy(et4.at[wid, b], et_v)

        def _gidx_row(j, _):
            for k in range(CHUNK // 16):
                sl = pl.ds(k * 16, 16)
                gidx_v[j, sl] = et_v[j, sl] * N_NODES + src_v[j, sl]
            return 0
        lax.fori_loop(0, RB, _gidx_row, 0)

        def _chunk(r, _):
            pltpu.async_copy(table.at[gidx_v.at[r]], rows_v, sem).wait()
            pltpu.sync_copy(rows_v, shared_acc.at[dst_v.at[r]], add=True)
            return 0
        lax.fori_loop(0, RB, _chunk, 0)
        return 0
    lax.fori_loop(0, NBLK, _block, 0)

    plsc.subcore_barrier()

    pltpu.sync_copy(shared_acc.at[pl.ds(base, NODES_PER_TILE)],
                    out.at[pl.ds(base, NODES_PER_TILE)])


def _sc_scatter(table2d, src4, dst4, et4):
    mesh = plsc.VectorSubcoreMesh(core_axis_name="c", subcore_axis_name="s",
                                  num_cores=NC)
    return pl.kernel(
        _sc_body,
        out_type=jax.ShapeDtypeStruct((N_PAD, D), jnp.float32),
        mesh=mesh,
        scratch_types=[
            pltpu.VMEM((RB, CHUNK), jnp.int32),
            pltpu.VMEM((RB, CHUNK), jnp.int32),
            pltpu.VMEM((RB, CHUNK), jnp.int32),
            pltpu.VMEM((RB, CHUNK), jnp.int32),
            pltpu.VMEM((CHUNK, D), jnp.float32),
            pltpu.VMEM_SHARED((N_PAD, D), jnp.float32),
            pltpu.SemaphoreType.DMA,
        ],
    )(table2d, src4, dst4, et4)


def _add_body(f_ref, p0_ref, o_ref):
    o_ref[...] = f_ref[...] + p0_ref[...]


def _final_add(feat, p0):
    BN = 1000
    NB = N_NODES // BN
    spec = pl.BlockSpec((BN, D), lambda n: (n, 0))
    return pl.pallas_call(
        _add_body,
        grid=(NB,),
        in_specs=[spec, spec],
        out_specs=spec,
        out_shape=jax.ShapeDtypeStruct((N_NODES, D), jnp.float32),
    )(feat, p0)


@jax.jit
def kernel(feat, edge_index, etypes, W):
    table = _typed_transform(feat, W).reshape(R * N_NODES, D)
    src4 = edge_index[0].reshape(NW, NBLK, RB, CHUNK)
    dst4 = edge_index[1].reshape(NW, NBLK, RB, CHUNK)
    et4 = etypes.reshape(NW, NBLK, RB, CHUNK).astype(jnp.int32)
    partial = _sc_scatter(table, src4, dst4, et4)
    return _final_add(feat, partial[:N_NODES])


# speedup vs baseline: 17.2845x; 1.3999x over previous
import functools

import jax
import jax.numpy as jnp
from jax import lax
from jax.experimental import pallas as pl
from jax.experimental.pallas import tpu as pltpu
from jax.experimental.pallas import tpu_sc as plsc

N_NODES = 10000
N_EDGES = 320000
D = 128
R = 8

NC = 1
NS = 16
NW = NC * NS

CHUNK = 80
RB = 10
NBLK = N_EDGES // (CHUNK * RB * NW)
N_PAD = 10240
NODES_PER_TILE = N_PAD // NS


def _mm_body(feat_ref, w_ref, out_ref):
    out_ref[0] = jnp.dot(feat_ref[...], w_ref[0],
                         preferred_element_type=jnp.float32)


def _typed_transform(feat, W):
    BN = 1000
    NB = N_NODES // BN
    return pl.pallas_call(
        _mm_body,
        grid=(NB, R),
        in_specs=[
            pl.BlockSpec((BN, D), lambda n, r: (n, 0)),
            pl.BlockSpec((1, D, D), lambda n, r: (r, 0, 0)),
        ],
        out_specs=pl.BlockSpec((1, BN, D), lambda n, r: (r, n, 0)),
        out_shape=jax.ShapeDtypeStruct((R, N_NODES, D), jnp.float32),
    )(feat, W)


def _sc_body(table, src4, dst4, et4, out, src_v, dst_v, et_v, gidx_v,
             rows_a, rows_b, shared_acc, sem_a, sem_b):
    s = lax.axis_index("s")
    wid = s
    base = s * NODES_PER_TILE
    bufs = (rows_a, rows_b)
    sems = (sem_a, sem_b)

    def _zero_row(j, _):
        for k in range(D // 16):
            rows_a[j, pl.ds(k * 16, 16)] = jnp.zeros((16,), jnp.float32)
        return 0
    lax.fori_loop(0, CHUNK, _zero_row, 0)
    for i in range(NODES_PER_TILE // CHUNK):
        pltpu.sync_copy(rows_a, shared_acc.at[pl.ds(base + i * CHUNK, CHUNK)])

    plsc.subcore_barrier()

    def _block(b, _):
        pltpu.sync_copy(src4.at[wid, b], src_v)
        pltpu.sync_copy(dst4.at[wid, b], dst_v)
        pltpu.sync_copy(et4.at[wid, b], et_v)

        def _gidx_row(j, _):
            for k in range(CHUNK // 16):
                sl = pl.ds(k * 16, 16)
                gidx_v[j, sl] = et_v[j, sl] * N_NODES + src_v[j, sl]
            return 0
        lax.fori_loop(0, RB, _gidx_row, 0)

        copies = [None] * RB
        copies[0] = pltpu.async_copy(table.at[gidx_v.at[0]], bufs[0], sems[0])
        for r in range(1, RB):
            copies[r] = pltpu.async_copy(table.at[gidx_v.at[r]],
                                         bufs[r % 2], sems[r % 2])
            copies[r - 1].wait()
            pltpu.sync_copy(bufs[(r - 1) % 2],
                            shared_acc.at[dst_v.at[r - 1]], add=True)
        copies[RB - 1].wait()
        pltpu.sync_copy(bufs[(RB - 1) % 2],
                        shared_acc.at[dst_v.at[RB - 1]], add=True)
        return 0
    lax.fori_loop(0, NBLK, _block, 0)

    plsc.subcore_barrier()

    pltpu.sync_copy(shared_acc.at[pl.ds(base, NODES_PER_TILE)],
                    out.at[pl.ds(base, NODES_PER_TILE)])


def _sc_scatter(table2d, src4, dst4, et4):
    mesh = plsc.VectorSubcoreMesh(core_axis_name="c", subcore_axis_name="s",
                                  num_cores=NC)
    return pl.kernel(
        _sc_body,
        out_type=jax.ShapeDtypeStruct((N_PAD, D), jnp.float32),
        mesh=mesh,
        scratch_types=[
            pltpu.VMEM((RB, CHUNK), jnp.int32),
            pltpu.VMEM((RB, CHUNK), jnp.int32),
            pltpu.VMEM((RB, CHUNK), jnp.int32),
            pltpu.VMEM((RB, CHUNK), jnp.int32),
            pltpu.VMEM((CHUNK, D), jnp.float32),
            pltpu.VMEM((CHUNK, D), jnp.float32),
            pltpu.VMEM_SHARED((N_PAD, D), jnp.float32),
            pltpu.SemaphoreType.DMA,
            pltpu.SemaphoreType.DMA,
        ],
    )(table2d, src4, dst4, et4)


def _add_body(f_ref, p0_ref, o_ref):
    o_ref[...] = f_ref[...] + p0_ref[...]


def _final_add(feat, p0):
    BN = 1000
    NB = N_NODES // BN
    spec = pl.BlockSpec((BN, D), lambda n: (n, 0))
    return pl.pallas_call(
        _add_body,
        grid=(NB,),
        in_specs=[spec, spec],
        out_specs=spec,
        out_shape=jax.ShapeDtypeStruct((N_NODES, D), jnp.float32),
    )(feat, p0)


@jax.jit
def kernel(feat, edge_index, etypes, W):
    table = _typed_transform(feat, W).reshape(R * N_NODES, D)
    src4 = edge_index[0].reshape(NW, NBLK, RB, CHUNK)
    dst4 = edge_index[1].reshape(NW, NBLK, RB, CHUNK)
    et4 = etypes.reshape(NW, NBLK, RB, CHUNK).astype(jnp.int32)
    partial = _sc_scatter(table, src4, dst4, et4)
    return _final_add(feat, partial[:N_NODES])
